# Initial kernel scaffold; baseline (speedup 1.0000x reference)
import jax
import jax.numpy as jnp
from jax.experimental import pallas as pl


def kernel(query, querychar, doc, docchar, neg, negchar, wordemb, charemb, w2c_w, w2c_b, qatt_k, qatt_o, datt_k, datt_o):
    raise NotImplementedError("write your pallas kernel here")


# speedup vs baseline: 2.1217x; 2.1217x over previous
import functools

import jax
import jax.numpy as jnp
from jax import lax
from jax.experimental import pallas as pl
from jax.experimental.pallas import tpu as pltpu
from jax.experimental.pallas import tpu_sc as plsc

_B = 4096
_QL = 20
_DL = 50
_WD = 100
_D = 128
_NWORD = 1000000
_VOCAB_C = 1000

_NC = 2
_NS = 16
_NW = _NC * _NS
_CHUNK = 128


def _proj_body(w_ref, wt_ref, b_ref, out_ref):
    out_ref[...] = lax.dot_general(
        w_ref[...], wt_ref[...], (((1,), (1,)), ((), ())),
        preferred_element_type=jnp.float32) + b_ref[...]


def _proj_call(rb):
    return pl.pallas_call(
        _proj_body,
        out_shape=jax.ShapeDtypeStruct((_NWORD, _D), jnp.float32),
        grid=(_NWORD // rb,),
        in_specs=[
            pl.BlockSpec((rb, _WD), lambda i: (i, 0)),
            pl.BlockSpec((_D, _WD), lambda i: (0, 0)),
            pl.BlockSpec((1, _D), lambda i: (0, 0)),
        ],
        out_specs=pl.BlockSpec((rb, _D), lambda i: (i, 0)),
    )


def _sc_gather_body(idx_q, idx_d, idx_n, table, out_q, out_d, out_n,
                    idx_v, rows_v, sem):
    wid = lax.axis_index("s") * _NC + lax.axis_index("c")

    def seg(idx_hbm, out_hbm, n_per_w):
        base = wid * n_per_w
        pltpu.sync_copy(idx_hbm.at[pl.ds(base, n_per_w)],
                        idx_v.at[pl.ds(0, n_per_w)])
        for c in range(n_per_w // _CHUNK):
            pltpu.async_copy(
                table.at[idx_v.at[pl.ds(c * _CHUNK, _CHUNK)]], rows_v,
                sem).wait()
            pltpu.sync_copy(rows_v,
                            out_hbm.at[pl.ds(base + c * _CHUNK, _CHUNK)])

    seg(idx_q, out_q, (_B * _QL) // _NW)
    seg(idx_d, out_d, (_B * _DL) // _NW)
    seg(idx_n, out_n, (_B * _DL) // _NW)


@functools.cache
def _sc_gather_call():
    return pl.kernel(
        _sc_gather_body,
        out_type=(
            jax.ShapeDtypeStruct((_B * _QL, _D), jnp.float32),
            jax.ShapeDtypeStruct((_B * _DL, _D), jnp.float32),
            jax.ShapeDtypeStruct((_B * _DL, _D), jnp.float32),
        ),
        mesh=plsc.VectorSubcoreMesh(core_axis_name="c", subcore_axis_name="s",
                                    num_cores=_NC),
        scratch_types=[
            pltpu.VMEM(((_B * _DL) // _NW,), jnp.int32),
            pltpu.VMEM((_CHUNK, _D), jnp.float32),
            pltpu.SemaphoreType.DMA,
        ],
    )


def _tower_body(rows_ref, cidx_ref, mid_ref, cemb_ref, attk_ref, atto_ref,
                out_ref, *, bb, seq, ch):
    t = bb * seq
    cemb16 = cemb_ref[...].astype(jnp.bfloat16)
    chunks = []
    for c in range(t // ch):
        cidx = cidx_ref[pl.ds(c * ch, ch), :]
        oh = (lax.broadcasted_iota(jnp.int32, (ch, _VOCAB_C), 1)
              == cidx).astype(jnp.bfloat16)
        chunks.append(jnp.dot(oh, cemb16,
                              preferred_element_type=jnp.float32))
    x = rows_ref[...] + jnp.concatenate(chunks, axis=0)

    th = jnp.tanh(lax.dot_general(x, attk_ref[...],
                                  (((1,), (1,)), ((), ())),
                                  preferred_element_type=jnp.float32))
    logits = jnp.sum(th * atto_ref[...], axis=1, keepdims=True)
    mask = (mid_ref[...] > 0).astype(jnp.float32)
    logits = logits - (1.0 - mask) * 1e12
    gmax = jnp.max(logits)
    ew = jnp.exp(logits - gmax)

    tdiv = lax.broadcasted_iota(jnp.int32, (bb, t), 1) // seq
    bidx = lax.broadcasted_iota(jnp.int32, (bb, t), 0)
    sel = (tdiv == bidx).astype(jnp.float32)
    xw = jnp.concatenate([x * ew, ew], axis=1)
    agg = jnp.dot(sel, xw, preferred_element_type=jnp.float32)
    den = jnp.maximum(agg[:, _D:_D + 1], 1e-30)
    pooled = agg[:, :_D] / den
    nrm = jnp.sqrt(jnp.sum(pooled * pooled, axis=1, keepdims=True))
    out_ref[...] = pooled / jnp.maximum(nrm, 1e-12)


def _tower_grid_spec(seq, bb):
    t = bb * seq
    return dict(
        grid=(_B // bb,),
        in_specs=[
            pl.BlockSpec((t, _D), lambda i: (i, 0)),
            pl.BlockSpec((t, 1), lambda i: (i, 0)),
            pl.BlockSpec((t, 1), lambda i: (i, 0)),
            pl.BlockSpec((_VOCAB_C, _D), lambda i: (0, 0)),
            pl.BlockSpec((_D, _D), lambda i: (0, 0)),
            pl.BlockSpec((1, _D), lambda i: (0, 0)),
        ],
        out_specs=pl.BlockSpec((bb, _D), lambda i: (i, 0)),
    )


def _tower_call(seq, bb, ch):
    spec = _tower_grid_spec(seq, bb)
    return pl.pallas_call(
        functools.partial(_tower_body, bb=bb, seq=seq, ch=ch),
        out_shape=jax.ShapeDtypeStruct((_B, _D), jnp.float32),
        **spec,
    )


def _loss_body(q_ref, d_ref, n_ref, out_ref, *, rb):
    i = pl.program_id(0)
    qb = q_ref[...]
    sd = 5.0 * lax.dot_general(qb, d_ref[...], (((1,), (1,)), ((), ())),
                               preferred_element_type=jnp.float32)
    sn = 5.0 * lax.dot_general(qb, n_ref[...], (((1,), (1,)), ((), ())),
                               preferred_element_type=jnp.float32)
    m = jnp.maximum(jnp.max(sd, axis=1, keepdims=True),
                    jnp.max(sn, axis=1, keepdims=True))
    ssum = (jnp.sum(jnp.exp(sd - m), axis=1, keepdims=True)
            + jnp.sum(jnp.exp(sn - m), axis=1, keepdims=True))
    lse = m + jnp.log(ssum)
    row = lax.broadcasted_iota(jnp.int32, (rb, _B), 0)
    col = lax.broadcasted_iota(jnp.int32, (rb, _B), 1)
    diag_sel = (col == row + i * rb).astype(jnp.float32)
    diag = jnp.sum(sd * diag_sel, axis=1, keepdims=True)
    partial = jnp.sum(lse - diag) / _B

    @pl.when(i == 0)
    def _():
        out_ref[...] = jnp.zeros_like(out_ref)

    out_ref[...] += partial


def _loss_grid_spec(rb):
    return dict(
        grid=(_B // rb,),
        in_specs=[
            pl.BlockSpec((rb, _D), lambda i: (i, 0)),
            pl.BlockSpec((_B, _D), lambda i: (0, 0)),
            pl.BlockSpec((_B, _D), lambda i: (0, 0)),
        ],
        out_specs=pl.BlockSpec((1, 1), lambda i: (0, 0)),
    )


def _loss_call(rb):
    spec = _loss_grid_spec(rb)
    return pl.pallas_call(
        functools.partial(_loss_body, rb=rb),
        out_shape=jax.ShapeDtypeStruct((1, 1), jnp.float32),
        **spec,
    )


def kernel(query, querychar, doc, docchar, neg, negchar, wordemb, charemb,
           w2c_w, w2c_b, qatt_k, qatt_o, datt_k, datt_o):
    iq = query.reshape(-1).astype(jnp.int32)
    idd = doc.reshape(-1).astype(jnp.int32)
    inn = neg.reshape(-1).astype(jnp.int32)

    ptable = _proj_call(8000)(wordemb, w2c_w, w2c_b.reshape(1, _D))
    qrows, drows, nrows = _sc_gather_call()(iq, idd, inn, ptable)

    qc = querychar.reshape(-1, 1).astype(jnp.int32)
    dc = docchar.reshape(-1, 1).astype(jnp.int32)
    nc = negchar.reshape(-1, 1).astype(jnp.int32)

    qemb = _tower_call(_QL, 128, 1280)(
        qrows, qc, qc, charemb, qatt_k, qatt_o)
    demb = _tower_call(_DL, 128, 1280)(
        drows, dc, idd.reshape(-1, 1), charemb, datt_k, datt_o)
    nemb = _tower_call(_DL, 128, 1280)(
        nrows, nc, inn.reshape(-1, 1), charemb, datt_k, datt_o)

    loss = _loss_call(512)(qemb, demb, nemb)
    return loss.reshape(())


# speedup vs baseline: 2.1277x; 1.0028x over previous
import functools

import jax
import jax.numpy as jnp
from jax import lax
from jax.experimental import pallas as pl
from jax.experimental.pallas import tpu as pltpu
from jax.experimental.pallas import tpu_sc as plsc

_B = 4096
_QL = 20
_DL = 50
_WD = 100
_D = 128
_NWORD = 1000000
_VOCAB_C = 1000

_NC = 2
_NS = 16
_NW = _NC * _NS
_CHUNK = 128


def _proj_body(w_ref, wt_ref, b_ref, out_ref):
    w16 = w_ref[...].astype(jnp.bfloat16)
    wt16 = wt_ref[...].astype(jnp.bfloat16)
    out_ref[...] = jnp.dot(w16, wt16,
                           preferred_element_type=jnp.float32) + b_ref[...]


def _proj_call(rb):
    return pl.pallas_call(
        _proj_body,
        out_shape=jax.ShapeDtypeStruct((_NWORD, _D), jnp.float32),
        grid=(_NWORD // rb,),
        in_specs=[
            pl.BlockSpec((rb, _WD), lambda i: (i, 0)),
            pl.BlockSpec((_WD, _D), lambda i: (0, 0)),
            pl.BlockSpec((1, _D), lambda i: (0, 0)),
        ],
        out_specs=pl.BlockSpec((rb, _D), lambda i: (i, 0)),
    )


def _sc_gather_body(idx_hbm, table, out_hbm, idx_v, rows0, rows1,
                    gsem0, gsem1, ssem0, ssem1, *, n_per_w):
    wid = lax.axis_index("s") * _NC + lax.axis_index("c")
    base = wid * n_per_w
    pltpu.sync_copy(idx_hbm.at[pl.ds(base, n_per_w)], idx_v)
    bufs = (rows0, rows1)
    gsems = (gsem0, gsem1)
    ssems = (ssem0, ssem1)
    nch = n_per_w // _CHUNK
    gathers = [None] * nch
    stores = [None] * nch
    for c in range(nch):
        p = c % 2
        if c >= 2:
            stores[c - 2].wait()
        gathers[c] = pltpu.async_copy(
            table.at[idx_v.at[pl.ds(c * _CHUNK, _CHUNK)]], bufs[p], gsems[p])
        if c >= 1:
            gathers[c - 1].wait()
            stores[c - 1] = pltpu.async_copy(
                bufs[(c - 1) % 2],
                out_hbm.at[pl.ds(base + (c - 1) * _CHUNK, _CHUNK)],
                ssems[(c - 1) % 2])
    gathers[nch - 1].wait()
    stores[nch - 1] = pltpu.async_copy(
        bufs[(nch - 1) % 2],
        out_hbm.at[pl.ds(base + (nch - 1) * _CHUNK, _CHUNK)],
        ssems[(nch - 1) % 2])
    stores[nch - 2].wait()
    stores[nch - 1].wait()


@functools.cache
def _sc_gather_call(n_tok):
    n_per_w = n_tok // _NW
    return pl.kernel(
        functools.partial(_sc_gather_body, n_per_w=n_per_w),
        out_type=jax.ShapeDtypeStruct((n_tok, _D), jnp.float32),
        mesh=plsc.VectorSubcoreMesh(core_axis_name="c", subcore_axis_name="s",
                                    num_cores=_NC),
        scratch_types=[
            pltpu.VMEM((n_per_w,), jnp.int32),
            pltpu.VMEM((_CHUNK, _D), jnp.float32),
            pltpu.VMEM((_CHUNK, _D), jnp.float32),
            pltpu.SemaphoreType.DMA,
            pltpu.SemaphoreType.DMA,
            pltpu.SemaphoreType.DMA,
            pltpu.SemaphoreType.DMA,
        ],
    )


def _tower_body(rows_ref, cidx_ref, mid_ref, cemb_ref, attk_ref, atto_ref,
                out_ref, *, bb, seq, ch):
    t = bb * seq
    cemb16 = cemb_ref[...].astype(jnp.bfloat16)
    chunks = []
    for c in range(t // ch):
        cidx = cidx_ref[pl.ds(c * ch, ch), :]
        oh = (lax.broadcasted_iota(jnp.int32, (ch, _VOCAB_C), 1)
              == cidx).astype(jnp.bfloat16)
        chunks.append(jnp.dot(oh, cemb16,
                              preferred_element_type=jnp.float32))
    x = rows_ref[...] + jnp.concatenate(chunks, axis=0)

    th = jnp.tanh(lax.dot_general(x, attk_ref[...],
                                  (((1,), (1,)), ((), ())),
                                  preferred_element_type=jnp.float32))
    logits = jnp.sum(th * atto_ref[...], axis=1, keepdims=True)
    mask = (mid_ref[...] > 0).astype(jnp.float32)
    logits = logits - (1.0 - mask) * 1e12
    gmax = jnp.max(logits)
    ew = jnp.exp(logits - gmax)

    tdiv = lax.broadcasted_iota(jnp.int32, (bb, t), 1) // seq
    bidx = lax.broadcasted_iota(jnp.int32, (bb, t), 0)
    sel = (tdiv == bidx).astype(jnp.float32)
    xw = jnp.concatenate([x * ew, ew], axis=1)
    agg = jnp.dot(sel, xw, preferred_element_type=jnp.float32)
    den = jnp.maximum(agg[:, _D:_D + 1], 1e-30)
    pooled = agg[:, :_D] / den
    nrm = jnp.sqrt(jnp.sum(pooled * pooled, axis=1, keepdims=True))
    out_ref[...] = pooled / jnp.maximum(nrm, 1e-12)


def _tower_grid_spec(seq, bb):
    t = bb * seq
    return dict(
        grid=(_B // bb,),
        in_specs=[
            pl.BlockSpec((t, _D), lambda i: (i, 0)),
            pl.BlockSpec((t, 1), lambda i: (i, 0)),
            pl.BlockSpec((t, 1), lambda i: (i, 0)),
            pl.BlockSpec((_VOCAB_C, _D), lambda i: (0, 0)),
            pl.BlockSpec((_D, _D), lambda i: (0, 0)),
            pl.BlockSpec((1, _D), lambda i: (0, 0)),
        ],
        out_specs=pl.BlockSpec((bb, _D), lambda i: (i, 0)),
    )


def _tower_call(seq, bb, ch):
    spec = _tower_grid_spec(seq, bb)
    return pl.pallas_call(
        functools.partial(_tower_body, bb=bb, seq=seq, ch=ch),
        out_shape=jax.ShapeDtypeStruct((_B, _D), jnp.float32),
        **spec,
    )


def _loss_body(q_ref, d_ref, n_ref, out_ref, *, rb):
    i = pl.program_id(0)
    qb = q_ref[...]
    sd = 5.0 * lax.dot_general(qb, d_ref[...], (((1,), (1,)), ((), ())),
                               preferred_element_type=jnp.float32)
    sn = 5.0 * lax.dot_general(qb, n_ref[...], (((1,), (1,)), ((), ())),
                               preferred_element_type=jnp.float32)
    m = jnp.maximum(jnp.max(sd, axis=1, keepdims=True),
                    jnp.max(sn, axis=1, keepdims=True))
    ssum = (jnp.sum(jnp.exp(sd - m), axis=1, keepdims=True)
            + jnp.sum(jnp.exp(sn - m), axis=1, keepdims=True))
    lse = m + jnp.log(ssum)
    row = lax.broadcasted_iota(jnp.int32, (rb, _B), 0)
    col = lax.broadcasted_iota(jnp.int32, (rb, _B), 1)
    diag_sel = (col == row + i * rb).astype(jnp.float32)
    diag = jnp.sum(sd * diag_sel, axis=1, keepdims=True)
    partial = jnp.sum(lse - diag) / _B

    @pl.when(i == 0)
    def _():
        out_ref[...] = jnp.zeros_like(out_ref)

    out_ref[...] += partial


def _loss_grid_spec(rb):
    return dict(
        grid=(_B // rb,),
        in_specs=[
            pl.BlockSpec((rb, _D), lambda i: (i, 0)),
            pl.BlockSpec((_B, _D), lambda i: (0, 0)),
            pl.BlockSpec((_B, _D), lambda i: (0, 0)),
        ],
        out_specs=pl.BlockSpec((1, 1), lambda i: (0, 0)),
    )


def _loss_call(rb):
    spec = _loss_grid_spec(rb)
    return pl.pallas_call(
        functools.partial(_loss_body, rb=rb),
        out_shape=jax.ShapeDtypeStruct((1, 1), jnp.float32),
        **spec,
    )


def kernel(query, querychar, doc, docchar, neg, negchar, wordemb, charemb,
           w2c_w, w2c_b, qatt_k, qatt_o, datt_k, datt_o):
    iq = query.reshape(-1).astype(jnp.int32)
    idd = doc.reshape(-1).astype(jnp.int32)
    inn = neg.reshape(-1).astype(jnp.int32)

    ptable = _proj_call(20000)(wordemb, w2c_w.T, w2c_b.reshape(1, _D))
    qrows = _sc_gather_call(_B * _QL)(iq, ptable)
    drows = _sc_gather_call(_B * _DL)(idd, ptable)
    nrows = _sc_gather_call(_B * _DL)(inn, ptable)

    qc = querychar.reshape(-1, 1).astype(jnp.int32)
    dc = docchar.reshape(-1, 1).astype(jnp.int32)
    nc = negchar.reshape(-1, 1).astype(jnp.int32)

    qemb = _tower_call(_QL, 128, 1280)(
        qrows, qc, qc, charemb, qatt_k, qatt_o)
    demb = _tower_call(_DL, 128, 1280)(
        drows, dc, idd.reshape(-1, 1), charemb, datt_k, datt_o)
    nemb = _tower_call(_DL, 128, 1280)(
        nrows, nc, inn.reshape(-1, 1), charemb, datt_k, datt_o)

    loss = _loss_call(512)(qemb, demb, nemb)
    return loss.reshape(())


# speedup vs baseline: 2.1868x; 1.0278x over previous
import functools

import jax
import jax.numpy as jnp
from jax import lax
from jax.experimental import pallas as pl
from jax.experimental.pallas import tpu as pltpu
from jax.experimental.pallas import tpu_sc as plsc

_B = 4096
_QL = 20
_DL = 50
_WD = 100
_D = 128
_NWORD = 1000000
_VOCAB_C = 1000

_NC = 2
_NS = 16
_NW = _NC * _NS
_CHUNK = 64

_NQ = _B * _QL
_NDN = 2 * _B * _DL


def _proj_body(w_ref, wt_ref, b_ref, out_ref):
    w16 = w_ref[...].astype(jnp.bfloat16)
    wt16 = wt_ref[...].astype(jnp.bfloat16)
    out_ref[...] = jnp.dot(w16, wt16,
                           preferred_element_type=jnp.float32) + b_ref[...]


def _proj_call(rb):
    return pl.pallas_call(
        _proj_body,
        out_shape=jax.ShapeDtypeStruct((_NWORD, _D), jnp.float32),
        grid=(_NWORD // rb,),
        in_specs=[
            pl.BlockSpec((rb, _WD), lambda i: (i, 0)),
            pl.BlockSpec((_WD, _D), lambda i: (0, 0)),
            pl.BlockSpec((1, _D), lambda i: (0, 0)),
        ],
        out_specs=pl.BlockSpec((rb, _D), lambda i: (i, 0)),
    )


def _sc_body(widx_q, cidx_q, widx_dn, cidx_dn, table, cemb, out_q, out_dn,
             idx_v, cdx_v, w0, w1, w2, w3, c0, c1, c2, c3, spm,
             gw0, gw1, gw2, gw3, gc0, gc1, gc2, gc3, ss0, ss1, ss2, ss3):
    sid = lax.axis_index("s")
    cid = lax.axis_index("c")
    wid = sid * _NC + cid

    @pl.when(sid == 0)
    def _():
        pltpu.sync_copy(cemb, spm)

    plsc.subcore_barrier()

    wbufs = (w0, w1, w2, w3)
    cbufs = (c0, c1, c2, c3)
    gwsems = (gw0, gw1, gw2, gw3)
    gcsems = (gc0, gc1, gc2, gc3)
    ssems = (ss0, ss1, ss2, ss3)

    def add_chunk(p):
        def body(r, _):
            for g in range(_D // 16):
                sl = pl.ds(g * 16, 16)
                wbufs[p][r, sl] += cbufs[p][r, sl]
            return ()
        lax.fori_loop(0, _CHUNK, body, (), unroll=4)

    def seg(widx, cidx, out, n_per_w):
        base = wid * n_per_w
        nch = n_per_w // _CHUNK
        pltpu.sync_copy(widx.at[pl.ds(base, n_per_w)],
                        idx_v.at[pl.ds(0, n_per_w)])
        pltpu.sync_copy(cidx.at[pl.ds(base, n_per_w)],
                        cdx_v.at[pl.ds(0, n_per_w)])

        def issue_gathers(c, t):
            sl = pl.ds(c * _CHUNK, _CHUNK)
            pltpu.async_copy(table.at[idx_v.at[sl]], wbufs[t], gwsems[t])
            pltpu.async_copy(spm.at[cdx_v.at[sl]], cbufs[t], gcsems[t])

        def wait_gathers(c, t):
            sl = pl.ds(c * _CHUNK, _CHUNK)
            pltpu.make_async_copy(table.at[idx_v.at[sl]], wbufs[t],
                                  gwsems[t]).wait()
            pltpu.make_async_copy(spm.at[cdx_v.at[sl]], cbufs[t],
                                  gcsems[t]).wait()

        def issue_store(c, t):
            pltpu.async_copy(wbufs[t],
                             out.at[pl.ds(base + c * _CHUNK, _CHUNK)],
                             ssems[t])

        def wait_store(c, t):
            pltpu.make_async_copy(wbufs[t],
                                  out.at[pl.ds(base + c * _CHUNK, _CHUNK)],
                                  ssems[t]).wait()

        def process(c, t):
            wait_gathers(c, t)
            add_chunk(t)
            issue_store(c, t)

        def quad(j, _):
            for t in range(4):
                c = 4 * j + t

                @pl.when(j > 0)
                def _():
                    wait_store(c - 4, t)
                issue_gathers(c, t)
                if t == 0:
                    @pl.when(j > 0)
                    def _():
                        process(c - 1, 3)
                else:
                    process(c - 1, t - 1)
            return ()

        lax.fori_loop(0, nch // 4, quad, ())
        process(nch - 1, 3)
        for t in range(4):
            wait_store(nch - 4 + t, t)

    seg(widx_q, cidx_q, out_q, _NQ // _NW)
    seg(widx_dn, cidx_dn, out_dn, _NDN // _NW)


@functools.cache
def _sc_gather_call():
    n_per_w = _NDN // _NW
    return pl.kernel(
        _sc_body,
        out_type=(
            jax.ShapeDtypeStruct((_NQ, _D), jnp.float32),
            jax.ShapeDtypeStruct((_NDN, _D), jnp.float32),
        ),
        mesh=plsc.VectorSubcoreMesh(core_axis_name="c", subcore_axis_name="s",
                                    num_cores=_NC),
        scratch_types=(
            [pltpu.VMEM((n_per_w,), jnp.int32)] * 2
            + [pltpu.VMEM((_CHUNK, _D), jnp.float32)] * 8
            + [pltpu.VMEM_SHARED((_VOCAB_C, _D), jnp.float32)]
            + [pltpu.SemaphoreType.DMA] * 12
        ),
    )


def _tower_body(rows_ref, mid_ref, attk_ref, atto_ref, out_ref, *, bb, seq):
    t = bb * seq
    x = rows_ref[...]
    th = jnp.tanh(lax.dot_general(x, attk_ref[...],
                                  (((1,), (1,)), ((), ())),
                                  preferred_element_type=jnp.float32))
    logits = jnp.sum(th * atto_ref[...], axis=1, keepdims=True)
    mask = (mid_ref[...] > 0).astype(jnp.float32)
    logits = logits - (1.0 - mask) * 1e12
    gmax = jnp.max(logits)
    ew = jnp.exp(logits - gmax)

    tdiv = lax.broadcasted_iota(jnp.int32, (bb, t), 1) // seq
    bidx = lax.broadcasted_iota(jnp.int32, (bb, t), 0)
    sel = (tdiv == bidx).astype(jnp.float32)
    xw = jnp.concatenate([x * ew, ew], axis=1)
    agg = jnp.dot(sel, xw, preferred_element_type=jnp.float32)
    den = jnp.maximum(agg[:, _D:_D + 1], 1e-30)
    pooled = agg[:, :_D] / den
    nrm = jnp.sqrt(jnp.sum(pooled * pooled, axis=1, keepdims=True))
    out_ref[...] = pooled / jnp.maximum(nrm, 1e-12)


def _tower_call(seq, bb, nb):
    t = bb * seq
    return pl.pallas_call(
        functools.partial(_tower_body, bb=bb, seq=seq),
        out_shape=jax.ShapeDtypeStruct((nb, _D), jnp.float32),
        grid=(nb // bb,),
        in_specs=[
            pl.BlockSpec((t, _D), lambda i: (i, 0)),
            pl.BlockSpec((t, 1), lambda i: (i, 0)),
            pl.BlockSpec((_D, _D), lambda i: (0, 0)),
            pl.BlockSpec((1, _D), lambda i: (0, 0)),
        ],
        out_specs=pl.BlockSpec((bb, _D), lambda i: (i, 0)),
    )


def _loss_body(q_ref, dn_ref, out_ref, *, rb):
    i = pl.program_id(0)
    qb = q_ref[...]
    s = 5.0 * lax.dot_general(qb, dn_ref[...], (((1,), (1,)), ((), ())),
                              preferred_element_type=jnp.float32)
    m = jnp.max(s, axis=1, keepdims=True)
    ssum = jnp.sum(jnp.exp(s - m), axis=1, keepdims=True)
    lse = m + jnp.log(ssum)
    dmatch = dn_ref[pl.ds(i * rb, rb), :]
    diag = 5.0 * jnp.sum(qb * dmatch, axis=1, keepdims=True)
    partial = jnp.sum(lse - diag) / _B

    @pl.when(i == 0)
    def _():
        out_ref[...] = jnp.zeros_like(out_ref)

    out_ref[...] += partial


def _loss_call(rb):
    return pl.pallas_call(
        functools.partial(_loss_body, rb=rb),
        out_shape=jax.ShapeDtypeStruct((1, 1), jnp.float32),
        grid=(_B // rb,),
        in_specs=[
            pl.BlockSpec((rb, _D), lambda i: (i, 0)),
            pl.BlockSpec((2 * _B, _D), lambda i: (0, 0)),
        ],
        out_specs=pl.BlockSpec((1, 1), lambda i: (0, 0)),
    )


def kernel(query, querychar, doc, docchar, neg, negchar, wordemb, charemb,
           w2c_w, w2c_b, qatt_k, qatt_o, datt_k, datt_o):
    iq = query.reshape(-1).astype(jnp.int32)
    idn = jnp.concatenate([doc.reshape(-1), neg.reshape(-1)]).astype(jnp.int32)
    cq = querychar.reshape(-1).astype(jnp.int32)
    cdn = jnp.concatenate([docchar.reshape(-1),
                           negchar.reshape(-1)]).astype(jnp.int32)

    ptable = _proj_call(20000)(wordemb, w2c_w.T, w2c_b.reshape(1, _D))
    qrows, dnrows = _sc_gather_call()(iq, cq, idn, cdn, ptable, charemb)

    qemb = _tower_call(_QL, 128, _B)(
        qrows, cq.reshape(-1, 1), qatt_k, qatt_o)
    dnemb = _tower_call(_DL, 128, 2 * _B)(
        dnrows, idn.reshape(-1, 1), datt_k, datt_o)

    loss = _loss_call(512)(qemb, dnemb)
    return loss.reshape(())
